# Initial kernel scaffold; baseline (speedup 1.0000x reference)
import jax
import jax.numpy as jnp
from jax.experimental import pallas as pl


def kernel(hidden_states, gate_w, w_gate, w_up, w_down):
    raise NotImplementedError("write your pallas kernel here")


# speedup vs baseline: 1.4632x; 1.4632x over previous
import functools

import jax
import jax.numpy as jnp
from jax.experimental import pallas as pl
from jax.experimental.pallas import tpu as pltpu

E = 8
TOPK = 2
TM = 256


def _router_combine(x, gate_w):
    logits = jax.lax.dot_general(
        x, gate_w, (((1,), (1,)), ((), ())),
        preferred_element_type=jnp.float32)
    logits = logits.astype(jnp.bfloat16).astype(jnp.float32)
    m = jnp.max(logits, axis=1, keepdims=True)
    ex = jnp.exp(logits - m)
    probs = ex / jnp.sum(ex, axis=1, keepdims=True)
    idx = jax.lax.broadcasted_iota(jnp.int32, probs.shape, 1)
    big = jnp.int32(E)
    m1 = jnp.max(probs, axis=1, keepdims=True)
    i1 = jnp.min(jnp.where(probs == m1, idx, big), axis=1, keepdims=True)
    probs2 = jnp.where(idx == i1, -1.0, probs)
    m2 = jnp.max(probs2, axis=1, keepdims=True)
    i2 = jnp.min(jnp.where(probs2 == m2, idx, big), axis=1, keepdims=True)
    s = m1 + m2
    c1 = (m1 / s).astype(jnp.bfloat16).astype(jnp.float32)
    c2 = (m2 / s).astype(jnp.bfloat16).astype(jnp.float32)
    zero = jnp.zeros_like(c1)
    return (jnp.where(idx == i1, c1, zero)
            + jnp.where(idx == i2, c2, zero))


def _moe_kernel(x_ref, gate_ref, wg_ref, wu_ref, wd_ref, out_ref,
                acc_ref, comb_ref, *, n_tiles):
    e = pl.program_id(0)
    i = pl.program_id(1)
    rows = pl.ds(i * TM, TM)
    x = x_ref[...]

    @pl.when(e == 0)
    def _():
        comb_ref[rows, :] = _router_combine(x, gate_ref[...])

    wg = wg_ref[0]
    wu = wu_ref[0]
    wd = wd_ref[0]
    g = jax.lax.dot_general(x, wg, (((1,), (1,)), ((), ())),
                            preferred_element_type=jnp.float32)
    u = jax.lax.dot_general(x, wu, (((1,), (1,)), ((), ())),
                            preferred_element_type=jnp.float32)
    g16 = g.astype(jnp.bfloat16)
    u16 = u.astype(jnp.bfloat16)
    sig = (1.0 / (1.0 + jnp.exp(-g16.astype(jnp.float32))))
    h = ((g16.astype(jnp.float32) * sig).astype(jnp.bfloat16) * u16)
    y = jax.lax.dot_general(h, wd, (((1,), (1,)), ((), ())),
                            preferred_element_type=jnp.float32)
    y16 = y.astype(jnp.bfloat16)
    comb = comb_ref[rows, :]
    lane = jax.lax.broadcasted_iota(jnp.int32, comb.shape, 1)
    ce = jnp.sum(jnp.where(lane == e, comb, jnp.zeros_like(comb)),
                 axis=1, keepdims=True).astype(jnp.bfloat16)
    contrib = (ce * y16).astype(jnp.float32)

    @pl.when(e == 0)
    def _():
        acc_ref[rows, :] = contrib

    @pl.when(e != 0)
    def _():
        acc_ref[rows, :] = acc_ref[rows, :] + contrib

    @pl.when(e == E - 1)
    def _():
        out_ref[...] = acc_ref[rows, :].astype(jnp.bfloat16)


def kernel(hidden_states, gate_w, w_gate, w_up, w_down):
    b, s, d = hidden_states.shape
    x = hidden_states.reshape(-1, d)
    t = x.shape[0]
    dff = w_gate.shape[1]
    n_tiles = t // TM

    out = pl.pallas_call(
        functools.partial(_moe_kernel, n_tiles=n_tiles),
        grid=(E, n_tiles),
        in_specs=[
            pl.BlockSpec((TM, d), lambda e, i: (i, 0)),
            pl.BlockSpec((E, d), lambda e, i: (0, 0)),
            pl.BlockSpec((1, dff, d), lambda e, i: (e, 0, 0)),
            pl.BlockSpec((1, dff, d), lambda e, i: (e, 0, 0)),
            pl.BlockSpec((1, d, dff), lambda e, i: (e, 0, 0)),
        ],
        out_specs=pl.BlockSpec((TM, d), lambda e, i: (i, 0)),
        out_shape=jax.ShapeDtypeStruct((t, d), jnp.bfloat16),
        scratch_shapes=[
            pltpu.VMEM((t, d), jnp.float32),
            pltpu.VMEM((t, E), jnp.float32),
        ],
        compiler_params=pltpu.CompilerParams(
            dimension_semantics=("arbitrary", "arbitrary")),
    )(x, gate_w, w_gate, w_up, w_down)
    return out.reshape(b, s, d)
